# Initial kernel scaffold; baseline (speedup 1.0000x reference)
import jax
import jax.numpy as jnp
from jax.experimental import pallas as pl

_K = 5


def _acc_body(gt_ref, pred_ref, out_ref):
    i = pl.program_id(0)
    nsteps = pl.num_programs(0)
    pred = pred_ref[...]
    g = gt_ref[...]
    rb, n = pred.shape
    col = jax.lax.broadcasted_iota(jnp.int32, (rb, n), 1)
    onehot = col == g
    v = jnp.max(jnp.where(onehot, pred, -jnp.inf), axis=1, keepdims=True)
    cnt_gt = jnp.sum((pred > v).astype(jnp.int32), axis=1)
    cnt_eq = jnp.sum(((pred == v) & (col < g)).astype(jnp.int32), axis=1)
    part = jnp.sum(((cnt_gt + cnt_eq) < _K).astype(jnp.float32))

    @pl.when(i == 0)
    def _():
        out_ref[0, 0] = 0.0

    out_ref[0, 0] += part


def kernel(pred, gt):
    b, n = pred.shape
    rb = 8
    grid = (b // rb,)
    out = pl.pallas_call(
        _acc_body,
        grid=grid,
        in_specs=[
            pl.BlockSpec((rb, 1), lambda i: (i, 0)),
            pl.BlockSpec((rb, n), lambda i: (i, 0)),
        ],
        out_specs=pl.BlockSpec((1, 1), lambda i: (0, 0)),
        out_shape=jax.ShapeDtypeStruct((1, 1), jnp.float32),
    )(gt.reshape(b, 1), pred)
    return out[0, 0] / b


# speedup vs baseline: 4.1097x; 4.1097x over previous
import jax
import jax.numpy as jnp
from jax.experimental import pallas as pl

_K = 5


def _acc_body(gt_ref, pred_ref, out_ref):
    i = pl.program_id(0)
    nsteps = pl.num_programs(0)
    pred = pred_ref[...]
    g = gt_ref[...]
    rb, n = pred.shape
    col = jax.lax.broadcasted_iota(jnp.int32, (rb, n), 1)
    onehot = col == g
    v = jnp.max(jnp.where(onehot, pred, -jnp.inf), axis=1, keepdims=True)
    cnt_gt = jnp.sum((pred > v).astype(jnp.int32), axis=1)
    cnt_eq = jnp.sum(((pred == v) & (col < g)).astype(jnp.int32), axis=1)
    part = jnp.sum(((cnt_gt + cnt_eq) < _K).astype(jnp.float32)).reshape(1, 1)

    @pl.when(i == 0)
    def _():
        out_ref[...] = jnp.zeros((1, 1), jnp.float32)

    out_ref[...] += part


def kernel(pred, gt):
    b, n = pred.shape
    rb = 8
    grid = (b // rb,)
    out = pl.pallas_call(
        _acc_body,
        grid=grid,
        in_specs=[
            pl.BlockSpec((rb, 1), lambda i: (i, 0)),
            pl.BlockSpec((rb, n), lambda i: (i, 0)),
        ],
        out_specs=pl.BlockSpec((1, 1), lambda i: (0, 0)),
        out_shape=jax.ShapeDtypeStruct((1, 1), jnp.float32),
    )(gt.reshape(b, 1), pred)
    return out[0, 0] / b
